# baseline (speedup 1.0000x reference)
import functools

import jax
import jax.numpy as jnp
from jax.experimental import pallas as pl

B, S, HID = 1, 2048, 1024
E, TOPK, FF = 8, 2, 512
NORM_MIN = 1e-12
T = B * S

_HIGH = jax.lax.Precision.HIGHEST


def _router_body(logits_in_ref, bias_ref, logits_ref, combine_ref):
    logits = logits_in_ref[...]
    m = jnp.max(logits, axis=1, keepdims=True)
    p = jnp.exp(logits - m)
    sm = p / jnp.sum(p, axis=1, keepdims=True)
    corrected = sm + bias_ref[...]
    col = jax.lax.broadcasted_iota(jnp.int32, (T, E), 1)
    m1 = jnp.max(corrected, axis=1, keepdims=True)
    a1 = jnp.min(jnp.where(corrected == m1, col, E), axis=1, keepdims=True)
    oh1 = col == a1
    c2 = jnp.where(oh1, -jnp.inf, corrected)
    m2 = jnp.max(c2, axis=1, keepdims=True)
    a2 = jnp.min(jnp.where(c2 == m2, col, E), axis=1, keepdims=True)
    oh2 = col == a2
    w1 = jnp.sum(jnp.where(oh1, sm, 0.0), axis=1, keepdims=True)
    w2 = jnp.sum(jnp.where(oh2, sm, 0.0), axis=1, keepdims=True)
    s = jnp.maximum(w1 + w2, NORM_MIN)
    combine = (jnp.where(oh1, w1, 0.0) + jnp.where(oh2, w2, 0.0)) / s
    logits_ref[...] = logits
    combine_ref[...] = combine


BT = 512
NT = T // BT


def _ffn_body(x_ref, gp_ref, up_ref, dp_ref, combine_ref, out_ref, acc_ref):
    e = pl.program_id(0)
    t = pl.program_id(1)
    rows = pl.ds(t * BT, BT)

    x = x_ref[...]
    g = jnp.dot(x, gp_ref[0], preferred_element_type=jnp.float32,
                precision=_HIGH)
    u = jnp.dot(x, up_ref[0], preferred_element_type=jnp.float32,
                precision=_HIGH)
    h = (g * jax.nn.sigmoid(g)) * u
    y = jnp.dot(h, dp_ref[0], preferred_element_type=jnp.float32,
                precision=_HIGH)
    col = jax.lax.broadcasted_iota(jnp.int32, (BT, E), 1)
    c = jnp.sum(jnp.where(col == e, combine_ref[...], 0.0), axis=1,
                keepdims=True)
    y = y * c

    @pl.when(e == 0)
    def _():
        acc_ref[rows, :] = y

    @pl.when((e != 0) & (e != E - 1))
    def _():
        acc_ref[rows, :] += y

    @pl.when(e == E - 1)
    def _():
        out_ref[...] = acc_ref[rows, :] + y


def kernel(hidden_states, gate_w, e_score_correction_bias, gate_proj,
           up_proj, down_proj):
    x = hidden_states.reshape(T, HID).astype(jnp.float32)
    logits_in = x @ gate_w.T

    logits, combine = pl.pallas_call(
        _router_body,
        out_shape=[
            jax.ShapeDtypeStruct((T, E), jnp.float32),
            jax.ShapeDtypeStruct((T, E), jnp.float32),
        ],
    )(logits_in, e_score_correction_bias)

    from jax.experimental.pallas import tpu as pltpu

    final = pl.pallas_call(
        _ffn_body,
        grid=(E, NT),
        in_specs=[
            pl.BlockSpec((BT, HID), lambda e, t: (t, 0)),
            pl.BlockSpec((1, HID, FF), lambda e, t: (e, 0, 0)),
            pl.BlockSpec((1, HID, FF), lambda e, t: (e, 0, 0)),
            pl.BlockSpec((1, FF, HID), lambda e, t: (e, 0, 0)),
            pl.BlockSpec((BT, E), lambda e, t: (t, 0)),
        ],
        out_specs=pl.BlockSpec((BT, HID), lambda e, t: (t, 0)),
        out_shape=jax.ShapeDtypeStruct((T, HID), jnp.float32),
        scratch_shapes=[pltpu.VMEM((T, HID), jnp.float32)],
    )(x, gate_proj, up_proj, down_proj, combine)

    return (final.reshape(-1), logits.reshape(-1))


# speedup vs baseline: 2.8279x; 2.8279x over previous
import functools

import jax
import jax.numpy as jnp
from jax.experimental import pallas as pl

B, S, HID = 1, 2048, 1024
E, TOPK, FF = 8, 2, 512
NORM_MIN = 1e-12
T = B * S

_HIGH = jax.lax.Precision.HIGHEST


def _router_body(logits_in_ref, bias_ref, logits_ref, combine_ref):
    logits = logits_in_ref[...]
    m = jnp.max(logits, axis=1, keepdims=True)
    p = jnp.exp(logits - m)
    sm = p / jnp.sum(p, axis=1, keepdims=True)
    corrected = sm + bias_ref[...]
    col = jax.lax.broadcasted_iota(jnp.int32, (T, E), 1)
    m1 = jnp.max(corrected, axis=1, keepdims=True)
    a1 = jnp.min(jnp.where(corrected == m1, col, E), axis=1, keepdims=True)
    oh1 = col == a1
    c2 = jnp.where(oh1, -jnp.inf, corrected)
    m2 = jnp.max(c2, axis=1, keepdims=True)
    a2 = jnp.min(jnp.where(c2 == m2, col, E), axis=1, keepdims=True)
    oh2 = col == a2
    w1 = jnp.sum(jnp.where(oh1, sm, 0.0), axis=1, keepdims=True)
    w2 = jnp.sum(jnp.where(oh2, sm, 0.0), axis=1, keepdims=True)
    s = jnp.maximum(w1 + w2, NORM_MIN)
    combine = (jnp.where(oh1, w1, 0.0) + jnp.where(oh2, w2, 0.0)) / s
    logits_ref[...] = logits
    combine_ref[...] = combine


BT = 512
NT = T // BT


def _ffn_body(x_ref, gp_ref, up_ref, dp_ref, combine_ref, out_ref, acc_ref):
    e = pl.program_id(0)
    t = pl.program_id(1)
    rows = pl.ds(t * BT, BT)

    x = x_ref[...]
    g = jnp.dot(x, gp_ref[0], preferred_element_type=jnp.float32)
    u = jnp.dot(x, up_ref[0], preferred_element_type=jnp.float32)
    h = (g * jax.nn.sigmoid(g)) * u
    y = jnp.dot(h, dp_ref[0], preferred_element_type=jnp.float32)
    col = jax.lax.broadcasted_iota(jnp.int32, (BT, E), 1)
    c = jnp.sum(jnp.where(col == e, combine_ref[...], 0.0), axis=1,
                keepdims=True)
    y = y * c

    @pl.when(e == 0)
    def _():
        acc_ref[rows, :] = y

    @pl.when((e != 0) & (e != E - 1))
    def _():
        acc_ref[rows, :] += y

    @pl.when(e == E - 1)
    def _():
        out_ref[...] = acc_ref[rows, :] + y


def kernel(hidden_states, gate_w, e_score_correction_bias, gate_proj,
           up_proj, down_proj):
    x = hidden_states.reshape(T, HID).astype(jnp.float32)
    logits_in = x @ gate_w.T

    logits, combine = pl.pallas_call(
        _router_body,
        out_shape=[
            jax.ShapeDtypeStruct((T, E), jnp.float32),
            jax.ShapeDtypeStruct((T, E), jnp.float32),
        ],
    )(logits_in, e_score_correction_bias)

    from jax.experimental.pallas import tpu as pltpu

    final = pl.pallas_call(
        _ffn_body,
        grid=(E, NT),
        in_specs=[
            pl.BlockSpec((BT, HID), lambda e, t: (t, 0)),
            pl.BlockSpec((1, HID, FF), lambda e, t: (e, 0, 0)),
            pl.BlockSpec((1, HID, FF), lambda e, t: (e, 0, 0)),
            pl.BlockSpec((1, FF, HID), lambda e, t: (e, 0, 0)),
            pl.BlockSpec((BT, E), lambda e, t: (t, 0)),
        ],
        out_specs=pl.BlockSpec((BT, HID), lambda e, t: (t, 0)),
        out_shape=jax.ShapeDtypeStruct((T, HID), jnp.float32),
        scratch_shapes=[pltpu.VMEM((T, HID), jnp.float32)],
    )(x, gate_proj, up_proj, down_proj, combine)

    return (final.reshape(-1), logits.reshape(-1))
